# baseline (speedup 1.0000x reference)
import functools

import jax
import jax.numpy as jnp
from jax import lax
from jax.experimental import pallas as pl
from jax.experimental.pallas import tpu as pltpu
from jax.experimental.pallas import tpu_sc as plsc

EMB_D = 128
BATCH = 16384
NUM_CORES = 2
NUM_SUBCORES = 16
NUM_WORKERS = NUM_CORES * NUM_SUBCORES
B_PER_W = BATCH // NUM_WORKERS
NSTAGE = 8
CHUNK = B_PER_W // NSTAGE


def _build():
    mesh = plsc.VectorSubcoreMesh(core_axis_name="c", subcore_axis_name="s")

    @functools.partial(
        pl.kernel,
        mesh=mesh,
        out_type=jax.ShapeDtypeStruct((BATCH, EMB_D), jnp.float32),
        scratch_types=(
            [pltpu.VMEM((CHUNK,), jnp.int32) for _ in range(NSTAGE)]
            + [
                pltpu.VMEM((NSTAGE, CHUNK, EMB_D), jnp.float32),
                pltpu.VMEM_SHARED((3, EMB_D), jnp.float32),
                pltpu.SemaphoreType.DMA,
                pltpu.SemaphoreType.DMA,
                pltpu.SemaphoreType.DMA,
            ]
        ),
    )
    def lookup_kernel(labels_hbm, table_hbm, out_hbm, *rest):
        idx = rest[:NSTAGE]
        rows_v, tab_sh, sem_in, sem_g, sem_out = rest[NSTAGE:]
        sid = lax.axis_index("s")
        wid = sid * NUM_CORES + lax.axis_index("c")
        base = wid * B_PER_W
        cp_labs = [
            pltpu.async_copy(
                labels_hbm.at[pl.ds(base + k * CHUNK, CHUNK)], idx[k], sem_in)
            for k in range(NSTAGE)
        ]

        @pl.when(sid == 0)
        def _():
            pltpu.sync_copy(table_hbm, tab_sh)

        for cp in cp_labs:
            cp.wait()
        plsc.subcore_barrier()
        gs = [
            pltpu.async_copy(tab_sh.at[idx[k]], rows_v.at[k], sem_g)
            for k in range(NSTAGE)
        ]
        outs = []
        for k in range(NSTAGE):
            gs[k].wait()
            outs.append(pltpu.async_copy(
                rows_v.at[k],
                out_hbm.at[pl.ds(base + k * CHUNK, CHUNK)], sem_out))
        for cp in outs:
            cp.wait()

    return lookup_kernel


_lookup = _build()


def kernel(labels, table):
    out = _lookup(labels, table)
    return out.reshape(BATCH, 1, EMB_D)


# speedup vs baseline: 1.0348x; 1.0348x over previous
import functools

import jax
import jax.numpy as jnp
from jax import lax
from jax.experimental import pallas as pl
from jax.experimental.pallas import tpu as pltpu
from jax.experimental.pallas import tpu_sc as plsc

EMB_D = 128
BATCH = 16384
NUM_CORES = 2
NUM_SUBCORES = 16
NUM_WORKERS = NUM_CORES * NUM_SUBCORES
B_PER_W = BATCH // NUM_WORKERS
NSTAGE = 4
CHUNK = B_PER_W // NSTAGE


def _build():
    mesh = plsc.VectorSubcoreMesh(core_axis_name="c", subcore_axis_name="s")

    @functools.partial(
        pl.kernel,
        mesh=mesh,
        out_type=jax.ShapeDtypeStruct((BATCH, EMB_D), jnp.float32),
        scratch_types=(
            [pltpu.VMEM((CHUNK,), jnp.int32) for _ in range(NSTAGE)]
            + [
                pltpu.VMEM((NSTAGE, CHUNK, EMB_D), jnp.float32),
                pltpu.VMEM_SHARED((3, EMB_D), jnp.float32),
                pltpu.SemaphoreType.DMA,
                pltpu.SemaphoreType.DMA,
                pltpu.SemaphoreType.DMA,
            ]
        ),
    )
    def lookup_kernel(labels_hbm, table_hbm, out_hbm, *rest):
        idx = rest[:NSTAGE]
        rows_v, tab_sh, sem_in, sem_g, sem_out = rest[NSTAGE:]
        sid = lax.axis_index("s")
        wid = sid * NUM_CORES + lax.axis_index("c")
        base = wid * B_PER_W
        cp_labs = [
            pltpu.async_copy(
                labels_hbm.at[pl.ds(base + k * CHUNK, CHUNK)], idx[k], sem_in)
            for k in range(NSTAGE)
        ]

        @pl.when(sid == 0)
        def _():
            pltpu.sync_copy(table_hbm, tab_sh)

        for cp in cp_labs:
            cp.wait()
        plsc.subcore_barrier()
        gs = [
            pltpu.async_copy(tab_sh.at[idx[k]], rows_v.at[k], sem_g)
            for k in range(NSTAGE)
        ]
        outs = []
        for k in range(NSTAGE):
            gs[k].wait()
            outs.append(pltpu.async_copy(
                rows_v.at[k],
                out_hbm.at[pl.ds(base + k * CHUNK, CHUNK)], sem_out))
        for cp in outs:
            cp.wait()

    return lookup_kernel


_lookup = _build()


def kernel(labels, table):
    out = _lookup(labels, table)
    return out.reshape(BATCH, 1, EMB_D)
